# Initial kernel scaffold; baseline (speedup 1.0000x reference)
import jax
import jax.numpy as jnp
from jax.experimental import pallas as pl


def kernel(x, edge_index, W1, b1, W2, b2):
    raise NotImplementedError("write your pallas kernel here")


# speedup vs baseline: 2.6831x; 2.6831x over previous
import jax
import jax.numpy as jnp
from jax import lax
from jax.experimental import pallas as pl
from jax.experimental.pallas import tpu as pltpu
from jax.experimental.pallas import tpu_sc as plsc

N = 10000
E = 320000
D = 128
NC = 2
NS = 16
NW = NC * NS
C = 128
NCHUNK = 2560
EPAD = NCHUNK * C
CPT = NCHUNK // NW
NP = 10240
RPT = NP // NS

_MESH = plsc.VectorSubcoreMesh(core_axis_name="c", subcore_axis_name="s",
                               num_cores=NC, num_subcores=NS)
_OUT = jax.ShapeDtypeStruct((NC, NP, D), jnp.float32)


def _sc_segment_sum(y, src_r, dst_r, zeros):
  scratch = [
      pltpu.VMEM((C,), jnp.int32),
      pltpu.VMEM((C,), jnp.int32),
      pltpu.VMEM((C, D), jnp.float32),
      pltpu.VMEM_SHARED((NP, D), jnp.float32),
      pltpu.SemaphoreType.DMA,
  ]

  def body(y_ref, src_ref, dst_ref, z_ref, out_ref, idx_s, idx_d, rows,
           acc_sp, sem):
    c = lax.axis_index("c")
    s = lax.axis_index("s")
    pltpu.sync_copy(z_ref, acc_sp.at[pl.ds(s * RPT, RPT)])
    plsc.subcore_barrier()

    wid = c * NS + s

    def chunk_body(t, carry):
      row = wid * CPT + t
      pltpu.sync_copy(src_ref.at[row], idx_s)
      pltpu.sync_copy(dst_ref.at[row], idx_d)
      pltpu.async_copy(y_ref.at[idx_s], rows, sem).wait()
      pltpu.sync_copy(rows, acc_sp.at[idx_d], add=True)
      return carry

    lax.fori_loop(0, CPT, chunk_body, 0)
    plsc.subcore_barrier()
    pltpu.sync_copy(acc_sp.at[pl.ds(s * RPT, RPT)],
                    out_ref.at[c, pl.ds(s * RPT, RPT)])

  f = pl.kernel(body, out_type=_OUT, mesh=_MESH, scratch_types=scratch)
  return f(y, src_r, dst_r, zeros)


def _sc_degree(dst_r, ones, zeros):
  scratch = [
      pltpu.VMEM((C,), jnp.int32),
      pltpu.VMEM((C, D), jnp.float32),
      pltpu.VMEM_SHARED((NP, D), jnp.float32),
  ]

  def body(dst_ref, ones_ref, z_ref, out_ref, idx_d, ones_v, acc_sp):
    c = lax.axis_index("c")
    s = lax.axis_index("s")
    pltpu.sync_copy(z_ref, acc_sp.at[pl.ds(s * RPT, RPT)])
    pltpu.sync_copy(ones_ref, ones_v)
    plsc.subcore_barrier()

    wid = c * NS + s

    def chunk_body(t, carry):
      row = wid * CPT + t
      pltpu.sync_copy(dst_ref.at[row], idx_d)
      pltpu.sync_copy(ones_v, acc_sp.at[idx_d], add=True)
      return carry

    lax.fori_loop(0, CPT, chunk_body, 0)
    plsc.subcore_barrier()
    pltpu.sync_copy(acc_sp.at[pl.ds(s * RPT, RPT)],
                    out_ref.at[c, pl.ds(s * RPT, RPT)])

  f = pl.kernel(body, out_type=_OUT, mesh=_MESH, scratch_types=scratch)
  return f(dst_r, ones, zeros)


_DOT = (((1,), (1,)), ((), ()))
_RB = 1000


def _mm_body(x_ref, w_ref, o_ref):
  o_ref[...] = lax.dot_general(x_ref[...], w_ref[...], _DOT,
                               preferred_element_type=jnp.float32)


def _matmul(x, w):
  return pl.pallas_call(
      _mm_body,
      grid=(N // _RB,),
      in_specs=[pl.BlockSpec((_RB, D), lambda i: (i, 0)),
                pl.BlockSpec((D, D), lambda i: (0, 0))],
      out_specs=pl.BlockSpec((_RB, D), lambda i: (i, 0)),
      out_shape=jax.ShapeDtypeStruct((N, D), jnp.float32),
  )(x, w)


def _degcol_body(dagg_ref, o_ref):
  o_ref[...] = jnp.maximum(dagg_ref[0, :, :1] + dagg_ref[1, :, :1], 1.0)


def _degcol(dagg):
  return pl.pallas_call(
      _degcol_body,
      grid=(N // _RB,),
      in_specs=[pl.BlockSpec((NC, _RB, D), lambda i: (0, i, 0))],
      out_specs=pl.BlockSpec((_RB, 1), lambda i: (i, 0)),
      out_shape=jax.ShapeDtypeStruct((N, 1), jnp.float32),
  )(dagg)


def _mid_body(y_ref, agg_ref, deg_ref, b_ref, w_ref, o_ref):
  a = agg_ref[0] + agg_ref[1]
  h = jnp.maximum(y_ref[...] + a / deg_ref[...] + b_ref[...], 0.0)
  o_ref[...] = lax.dot_general(h, w_ref[...], _DOT,
                               preferred_element_type=jnp.float32)


def _mid(y, agg, deg, b, w):
  return pl.pallas_call(
      _mid_body,
      grid=(N // _RB,),
      in_specs=[pl.BlockSpec((_RB, D), lambda i: (i, 0)),
                pl.BlockSpec((NC, _RB, D), lambda i: (0, i, 0)),
                pl.BlockSpec((_RB, 1), lambda i: (i, 0)),
                pl.BlockSpec((1, D), lambda i: (0, 0)),
                pl.BlockSpec((D, D), lambda i: (0, 0))],
      out_specs=pl.BlockSpec((_RB, D), lambda i: (i, 0)),
      out_shape=jax.ShapeDtypeStruct((N, D), jnp.float32),
  )(y, agg, deg, b, w)


def _fin_body(y_ref, agg_ref, deg_ref, b_ref, o_ref):
  a = agg_ref[0] + agg_ref[1]
  o_ref[...] = y_ref[...] + a / deg_ref[...] + b_ref[...]


def _fin(y, agg, deg, b):
  return pl.pallas_call(
      _fin_body,
      grid=(N // _RB,),
      in_specs=[pl.BlockSpec((_RB, D), lambda i: (i, 0)),
                pl.BlockSpec((NC, _RB, D), lambda i: (0, i, 0)),
                pl.BlockSpec((_RB, 1), lambda i: (i, 0)),
                pl.BlockSpec((1, D), lambda i: (0, 0))],
      out_specs=pl.BlockSpec((_RB, D), lambda i: (i, 0)),
      out_shape=jax.ShapeDtypeStruct((N, D), jnp.float32),
  )(y, agg, deg, b)


def kernel(x, edge_index, W1, b1, W2, b2):
  src = edge_index[0]
  dst = edge_index[1]
  pad_s = jnp.zeros((EPAD - E,), jnp.int32)
  pad_d = jnp.full((EPAD - E,), N, jnp.int32)
  src_r = jnp.concatenate([src, pad_s]).reshape(NCHUNK, C)
  dst_r = jnp.concatenate([dst, pad_d]).reshape(NCHUNK, C)
  zeros = jnp.zeros((RPT, D), jnp.float32)
  ones = jnp.ones((C, D), jnp.float32)

  dagg = _sc_degree(dst_r, ones, zeros)
  deg = _degcol(dagg)
  y1 = _matmul(x, W1)
  agg1 = _sc_segment_sum(y1, src_r, dst_r, zeros)
  y2 = _mid(y1, agg1, deg, b1.reshape(1, D), W2)
  agg2 = _sc_segment_sum(y2, src_r, dst_r, zeros)
  return _fin(y2, agg2, deg, b2.reshape(1, D))


# speedup vs baseline: 3.4855x; 1.2990x over previous
import jax
import jax.numpy as jnp
from jax import lax
from jax.experimental import pallas as pl
from jax.experimental.pallas import tpu as pltpu
from jax.experimental.pallas import tpu_sc as plsc

N = 10000
E = 320000
D = 128
NC = 2
NS = 16
NW = NC * NS
C = 128
NCHUNK = 2560
EPAD = NCHUNK * C
CPT = NCHUNK // NW
NP = 10240
RPT = NP // NS

_MESH = plsc.VectorSubcoreMesh(core_axis_name="c", subcore_axis_name="s",
                               num_cores=NC, num_subcores=NS)
_OUT = jax.ShapeDtypeStruct((NC, NP, D), jnp.float32)

NSLOT = 2
HC = CPT // 2
NQH = HC // NSLOT


def _sc_segment_sum(y, src_r, dst_r, zeros):
  scratch = [
      pltpu.VMEM((HC, C), jnp.int32),
      pltpu.VMEM((HC, C), jnp.int32),
      pltpu.VMEM((NSLOT, C, D), jnp.float32),
      pltpu.VMEM_SHARED((NP, D), jnp.float32),
  ] + [pltpu.SemaphoreType.DMA] * NSLOT

  def body(y_ref, src_ref, dst_ref, z_ref, out_ref, idx_s, idx_d, rows,
           acc_sp, *gsem):
    c = lax.axis_index("c")
    s = lax.axis_index("s")
    pltpu.sync_copy(z_ref, acc_sp.at[pl.ds(s * RPT, RPT)])
    wid = c * NS + s
    base = wid * CPT

    def fire_gather(b, t):
      pltpu.async_copy(y_ref.at[idx_s.at[t]], rows.at[b], gsem[b])

    def drain_gather(b, t):
      pltpu.make_async_copy(y_ref.at[idx_s.at[t]], rows.at[b],
                            gsem[b]).wait()

    for half in range(2):
      pltpu.sync_copy(src_ref.at[pl.ds(base + half * HC, HC)], idx_s)
      pltpu.sync_copy(dst_ref.at[pl.ds(base + half * HC, HC)], idx_d)
      if half == 0:
        plsc.subcore_barrier()

      for b in range(NSLOT):
        fire_gather(b, b)

      def round_body(q, carry):
        t0 = q * NSLOT
        for b in range(NSLOT):
          drain_gather(b, t0 + b)
          pltpu.sync_copy(rows.at[b], acc_sp.at[idx_d.at[t0 + b]], add=True)
          fire_gather(b, t0 + NSLOT + b)
        return carry

      lax.fori_loop(0, NQH - 1, round_body, 0)

      t0 = (NQH - 1) * NSLOT
      for b in range(NSLOT):
        drain_gather(b, t0 + b)
        pltpu.sync_copy(rows.at[b], acc_sp.at[idx_d.at[t0 + b]], add=True)

    plsc.subcore_barrier()
    pltpu.sync_copy(acc_sp.at[pl.ds(s * RPT, RPT)],
                    out_ref.at[c, pl.ds(s * RPT, RPT)])

  f = pl.kernel(body, out_type=_OUT, mesh=_MESH, scratch_types=scratch)
  return f(y, src_r, dst_r, zeros)


_DSLOT = 4


def _sc_degree(dst_r, ones, zeros):
  scratch = [
      pltpu.VMEM((CPT, C), jnp.int32),
      pltpu.VMEM((C, D), jnp.float32),
      pltpu.VMEM_SHARED((NP, D), jnp.float32),
  ] + [pltpu.SemaphoreType.DMA] * _DSLOT

  def body(dst_ref, ones_ref, z_ref, out_ref, idx_d, ones_v, acc_sp, *ssem):
    c = lax.axis_index("c")
    s = lax.axis_index("s")
    pltpu.sync_copy(z_ref, acc_sp.at[pl.ds(s * RPT, RPT)])
    pltpu.sync_copy(ones_ref, ones_v)
    wid = c * NS + s
    base = wid * CPT
    pltpu.sync_copy(dst_ref.at[pl.ds(base, CPT)], idx_d)
    plsc.subcore_barrier()

    def quad_body(q, carry):
      t0 = q * _DSLOT
      hs = [pltpu.async_copy(ones_v, acc_sp.at[idx_d.at[t0 + b]],
                             ssem[b], add=True) for b in range(_DSLOT)]
      for h in hs:
        h.wait()
      return carry

    lax.fori_loop(0, CPT // _DSLOT, quad_body, 0)
    plsc.subcore_barrier()
    pltpu.sync_copy(acc_sp.at[pl.ds(s * RPT, RPT)],
                    out_ref.at[c, pl.ds(s * RPT, RPT)])

  f = pl.kernel(body, out_type=_OUT, mesh=_MESH, scratch_types=scratch)
  return f(dst_r, ones, zeros)


_DOT = (((1,), (1,)), ((), ()))
_RB = 1000


def _mm_body(x_ref, w_ref, o_ref):
  o_ref[...] = lax.dot_general(x_ref[...], w_ref[...], _DOT,
                               preferred_element_type=jnp.float32)


def _matmul(x, w):
  return pl.pallas_call(
      _mm_body,
      grid=(N // _RB,),
      in_specs=[pl.BlockSpec((_RB, D), lambda i: (i, 0)),
                pl.BlockSpec((D, D), lambda i: (0, 0))],
      out_specs=pl.BlockSpec((_RB, D), lambda i: (i, 0)),
      out_shape=jax.ShapeDtypeStruct((N, D), jnp.float32),
  )(x, w)


def _degcol_body(dagg_ref, o_ref):
  o_ref[...] = jnp.maximum(dagg_ref[0, :, :1] + dagg_ref[1, :, :1], 1.0)


def _degcol(dagg):
  return pl.pallas_call(
      _degcol_body,
      grid=(N // _RB,),
      in_specs=[pl.BlockSpec((NC, _RB, D), lambda i: (0, i, 0))],
      out_specs=pl.BlockSpec((_RB, 1), lambda i: (i, 0)),
      out_shape=jax.ShapeDtypeStruct((N, 1), jnp.float32),
  )(dagg)


def _mid_body(y_ref, agg_ref, deg_ref, b_ref, w_ref, o_ref):
  a = agg_ref[0] + agg_ref[1]
  h = jnp.maximum(y_ref[...] + a / deg_ref[...] + b_ref[...], 0.0)
  o_ref[...] = lax.dot_general(h, w_ref[...], _DOT,
                               preferred_element_type=jnp.float32)


def _mid(y, agg, deg, b, w):
  return pl.pallas_call(
      _mid_body,
      grid=(N // _RB,),
      in_specs=[pl.BlockSpec((_RB, D), lambda i: (i, 0)),
                pl.BlockSpec((NC, _RB, D), lambda i: (0, i, 0)),
                pl.BlockSpec((_RB, 1), lambda i: (i, 0)),
                pl.BlockSpec((1, D), lambda i: (0, 0)),
                pl.BlockSpec((D, D), lambda i: (0, 0))],
      out_specs=pl.BlockSpec((_RB, D), lambda i: (i, 0)),
      out_shape=jax.ShapeDtypeStruct((N, D), jnp.float32),
  )(y, agg, deg, b, w)


def _fin_body(y_ref, agg_ref, deg_ref, b_ref, o_ref):
  a = agg_ref[0] + agg_ref[1]
  o_ref[...] = y_ref[...] + a / deg_ref[...] + b_ref[...]


def _fin(y, agg, deg, b):
  return pl.pallas_call(
      _fin_body,
      grid=(N // _RB,),
      in_specs=[pl.BlockSpec((_RB, D), lambda i: (i, 0)),
                pl.BlockSpec((NC, _RB, D), lambda i: (0, i, 0)),
                pl.BlockSpec((_RB, 1), lambda i: (i, 0)),
                pl.BlockSpec((1, D), lambda i: (0, 0))],
      out_specs=pl.BlockSpec((_RB, D), lambda i: (i, 0)),
      out_shape=jax.ShapeDtypeStruct((N, D), jnp.float32),
  )(y, agg, deg, b)


def kernel(x, edge_index, W1, b1, W2, b2):
  src = edge_index[0]
  dst = edge_index[1]
  pad_s = jnp.zeros((EPAD - E,), jnp.int32)
  pad_d = jnp.full((EPAD - E,), N, jnp.int32)
  src_r = jnp.concatenate([src, pad_s]).reshape(NCHUNK, C)
  dst_r = jnp.concatenate([dst, pad_d]).reshape(NCHUNK, C)
  zeros = jnp.zeros((RPT, D), jnp.float32)
  ones = jnp.ones((C, D), jnp.float32)

  dagg = _sc_degree(dst_r, ones, zeros)
  deg = _degcol(dagg)
  y1 = _matmul(x, W1)
  agg1 = _sc_segment_sum(y1, src_r, dst_r, zeros)
  y2 = _mid(y1, agg1, deg, b1.reshape(1, D), W2)
  agg2 = _sc_segment_sum(y2, src_r, dst_r, zeros)
  return _fin(y2, agg2, deg, b2.reshape(1, D))


# speedup vs baseline: 3.6383x; 1.0439x over previous
import jax
import jax.numpy as jnp
from jax import lax
from jax.experimental import pallas as pl
from jax.experimental.pallas import tpu as pltpu
from jax.experimental.pallas import tpu_sc as plsc

N = 10000
E = 320000
D = 128
NC = 2
NS = 16
NW = NC * NS
C = 128
NCHUNK = 2560
EPAD = NCHUNK * C
CPT = NCHUNK // NW
NP = 10240
RPT = NP // NS

_MESH = plsc.VectorSubcoreMesh(core_axis_name="c", subcore_axis_name="s",
                               num_cores=NC, num_subcores=NS)
_OUT = jax.ShapeDtypeStruct((NC, NP, D), jnp.float32)

NSLOT = 2
HC = CPT // 2
NQH = HC // NSLOT
_FAST_CORE = 0


def _sc_segment_sum(y, src_r, dst_r, zeros):
  scratch = [
      pltpu.VMEM((HC, C), jnp.int32),
      pltpu.VMEM((HC, C), jnp.int32),
      pltpu.VMEM((NSLOT, C, D), jnp.float32),
      pltpu.VMEM_SHARED((NP, D), jnp.float32),
  ] + [pltpu.SemaphoreType.DMA] * NSLOT

  def body(y_ref, src_ref, dst_ref, z_ref, out_ref, idx_s, idx_d, rows,
           acc_sp, *gsem):
    c = lax.axis_index("c")
    s = lax.axis_index("s")
    pltpu.sync_copy(z_ref, acc_sp.at[pl.ds(s * RPT, RPT)])
    fast = jnp.int32(c == _FAST_CORE)
    nh = jnp.where(fast == 1, 3, 1)
    base = jnp.where(fast == 1, s * (3 * HC), 3 * HC * NS + s * HC)

    def fire_gather(b, t):
      pltpu.async_copy(y_ref.at[idx_s.at[t]], rows.at[b], gsem[b])

    def drain_gather(b, t):
      pltpu.make_async_copy(y_ref.at[idx_s.at[t]], rows.at[b],
                            gsem[b]).wait()

    plsc.subcore_barrier()

    def half_body(half, carry):
      pltpu.sync_copy(src_ref.at[pl.ds(base + half * HC, HC)], idx_s)
      pltpu.sync_copy(dst_ref.at[pl.ds(base + half * HC, HC)], idx_d)

      for b in range(NSLOT):
        fire_gather(b, b)

      def round_body(q, inner):
        t0 = q * NSLOT
        for b in range(NSLOT):
          drain_gather(b, t0 + b)
          pltpu.sync_copy(rows.at[b], acc_sp.at[idx_d.at[t0 + b]], add=True)
          fire_gather(b, t0 + NSLOT + b)
        return inner

      lax.fori_loop(0, NQH - 1, round_body, 0)

      t0 = (NQH - 1) * NSLOT
      for b in range(NSLOT):
        drain_gather(b, t0 + b)
        pltpu.sync_copy(rows.at[b], acc_sp.at[idx_d.at[t0 + b]], add=True)
      return carry

    lax.fori_loop(0, nh, half_body, 0)

    plsc.subcore_barrier()
    pltpu.sync_copy(acc_sp.at[pl.ds(s * RPT, RPT)],
                    out_ref.at[c, pl.ds(s * RPT, RPT)])

  f = pl.kernel(body, out_type=_OUT, mesh=_MESH, scratch_types=scratch)
  return f(y, src_r, dst_r, zeros)


_DSLOT = 4


def _sc_degree(dst_r, ones, zeros):
  scratch = [
      pltpu.VMEM((CPT, C), jnp.int32),
      pltpu.VMEM((C, D), jnp.float32),
      pltpu.VMEM_SHARED((NP, D), jnp.float32),
  ] + [pltpu.SemaphoreType.DMA] * _DSLOT

  def body(dst_ref, ones_ref, z_ref, out_ref, idx_d, ones_v, acc_sp, *ssem):
    c = lax.axis_index("c")
    s = lax.axis_index("s")
    pltpu.sync_copy(z_ref, acc_sp.at[pl.ds(s * RPT, RPT)])
    pltpu.sync_copy(ones_ref, ones_v)
    wid = c * NS + s
    base = wid * CPT
    pltpu.sync_copy(dst_ref.at[pl.ds(base, CPT)], idx_d)
    plsc.subcore_barrier()

    def quad_body(q, carry):
      t0 = q * _DSLOT
      hs = [pltpu.async_copy(ones_v, acc_sp.at[idx_d.at[t0 + b]],
                             ssem[b], add=True) for b in range(_DSLOT)]
      for h in hs:
        h.wait()
      return carry

    lax.fori_loop(0, CPT // _DSLOT, quad_body, 0)
    plsc.subcore_barrier()
    pltpu.sync_copy(acc_sp.at[pl.ds(s * RPT, RPT)],
                    out_ref.at[c, pl.ds(s * RPT, RPT)])

  f = pl.kernel(body, out_type=_OUT, mesh=_MESH, scratch_types=scratch)
  return f(dst_r, ones, zeros)


_DOT = (((1,), (1,)), ((), ()))
_RB = 1000


def _mm_body(x_ref, w_ref, o_ref):
  o_ref[...] = lax.dot_general(x_ref[...], w_ref[...], _DOT,
                               preferred_element_type=jnp.float32)


def _matmul(x, w):
  return pl.pallas_call(
      _mm_body,
      grid=(N // _RB,),
      in_specs=[pl.BlockSpec((_RB, D), lambda i: (i, 0)),
                pl.BlockSpec((D, D), lambda i: (0, 0))],
      out_specs=pl.BlockSpec((_RB, D), lambda i: (i, 0)),
      out_shape=jax.ShapeDtypeStruct((N, D), jnp.float32),
  )(x, w)


def _degcol_body(dagg_ref, o_ref):
  o_ref[...] = jnp.maximum(dagg_ref[0, :, :1] + dagg_ref[1, :, :1], 1.0)


def _degcol(dagg):
  return pl.pallas_call(
      _degcol_body,
      grid=(N // _RB,),
      in_specs=[pl.BlockSpec((NC, _RB, D), lambda i: (0, i, 0))],
      out_specs=pl.BlockSpec((_RB, 1), lambda i: (i, 0)),
      out_shape=jax.ShapeDtypeStruct((N, 1), jnp.float32),
  )(dagg)


def _mid_body(y_ref, agg_ref, deg_ref, b_ref, w_ref, o_ref):
  a = agg_ref[0] + agg_ref[1]
  h = jnp.maximum(y_ref[...] + a / deg_ref[...] + b_ref[...], 0.0)
  o_ref[...] = lax.dot_general(h, w_ref[...], _DOT,
                               preferred_element_type=jnp.float32)


def _mid(y, agg, deg, b, w):
  return pl.pallas_call(
      _mid_body,
      grid=(N // _RB,),
      in_specs=[pl.BlockSpec((_RB, D), lambda i: (i, 0)),
                pl.BlockSpec((NC, _RB, D), lambda i: (0, i, 0)),
                pl.BlockSpec((_RB, 1), lambda i: (i, 0)),
                pl.BlockSpec((1, D), lambda i: (0, 0)),
                pl.BlockSpec((D, D), lambda i: (0, 0))],
      out_specs=pl.BlockSpec((_RB, D), lambda i: (i, 0)),
      out_shape=jax.ShapeDtypeStruct((N, D), jnp.float32),
  )(y, agg, deg, b, w)


def _fin_body(y_ref, agg_ref, deg_ref, b_ref, o_ref):
  a = agg_ref[0] + agg_ref[1]
  o_ref[...] = y_ref[...] + a / deg_ref[...] + b_ref[...]


def _fin(y, agg, deg, b):
  return pl.pallas_call(
      _fin_body,
      grid=(N // _RB,),
      in_specs=[pl.BlockSpec((_RB, D), lambda i: (i, 0)),
                pl.BlockSpec((NC, _RB, D), lambda i: (0, i, 0)),
                pl.BlockSpec((_RB, 1), lambda i: (i, 0)),
                pl.BlockSpec((1, D), lambda i: (0, 0))],
      out_specs=pl.BlockSpec((_RB, D), lambda i: (i, 0)),
      out_shape=jax.ShapeDtypeStruct((N, D), jnp.float32),
  )(y, agg, deg, b)


def kernel(x, edge_index, W1, b1, W2, b2):
  src = edge_index[0]
  dst = edge_index[1]
  pad_s = jnp.zeros((EPAD - E,), jnp.int32)
  pad_d = jnp.full((EPAD - E,), N, jnp.int32)
  src_r = jnp.concatenate([src, pad_s]).reshape(NCHUNK, C)
  dst_r = jnp.concatenate([dst, pad_d]).reshape(NCHUNK, C)
  zeros = jnp.zeros((RPT, D), jnp.float32)
  ones = jnp.ones((C, D), jnp.float32)

  dagg = _sc_degree(dst_r, ones, zeros)
  deg = _degcol(dagg)
  y1 = _matmul(x, W1)
  agg1 = _sc_segment_sum(y1, src_r, dst_r, zeros)
  y2 = _mid(y1, agg1, deg, b1.reshape(1, D), W2)
  agg2 = _sc_segment_sum(y2, src_r, dst_r, zeros)
  return _fin(y2, agg2, deg, b2.reshape(1, D))
